# baseline (speedup 1.0000x reference)
import functools

import jax
import jax.numpy as jnp
from jax import lax
from jax.experimental import pallas as pl
from jax.experimental.pallas import tpu as pltpu
from jax.experimental.pallas import tpu_sc as plsc

_SEQ_LENS = (2048, 1792, 1536, 1280, 1024, 896, 768, 512)
_D = 1024
_MAX_L = 2048
_NC, _NS = 2, 16
_NW = _NC * _NS
_CH = 40
_NBUF = 3
_PCH = 128
_TOTAL_ROWS = 8 * _MAX_L


def _build_plans():
    segs = []
    for i, L in enumerate(_SEQ_LENS):
        segs.append(("data", i, 0, L))
        if L < _MAX_L:
            segs.append(("pad", i, L, _MAX_L))
    total_w = sum((r1 - r0) * (2 if k == "data" else 1) for k, _, r0, r1 in segs)
    target = total_w / _NW
    plans = [{"data": [], "pad": []} for _ in range(_NW)]
    acc = 0.0
    for kind, i, r0, r1 in segs:
        wpr = 2 if kind == "data" else 1
        r = r0
        while r < r1:
            w = min(_NW - 1, int(acc / target))
            rows = min(8, r1 - r)
            lst = plans[w][kind]
            if lst and lst[-1][0] == i and lst[-1][1] + lst[-1][2] == r:
                lst[-1] = (i, lst[-1][1], lst[-1][2] + rows)
            else:
                lst.append((i, r, rows))
            acc += rows * wpr
            r += rows
    return [( [tuple(x) for x in p["data"]], [tuple(x) for x in p["pad"]] )
            for p in plans]


_PLANS = _build_plans()


def _emit_worker(w, in_refs, out_ref, bufs, zsh, rd_sems, wr_sems, pad_sem):
    data_segs, pad_segs = _PLANS[w]

    chunks = []
    k = 0
    for i, r0, rows in data_segs:
        r = r0
        while r < r0 + rows:
            n = min(_CH, r0 + rows - r)
            chunks.append((k % _NBUF, i, r, i * _MAX_L + r, n))
            k += 1
            r += n
    n_chunks = len(chunks)

    rd = [None] * n_chunks
    wr = [None] * n_chunks

    def start_read(c):
        b, i, sr, dr, n = chunks[c]
        rd[c] = pltpu.async_copy(
            in_refs[i].at[pl.ds(sr, n)], bufs[b].at[pl.ds(0, n)], rd_sems.at[b])

    for c in range(min(_NBUF, n_chunks)):
        start_read(c)

    for c in range(n_chunks):
        b, i, sr, dr, n = chunks[c]
        rd[c].wait()
        wr[c] = pltpu.async_copy(
            bufs[b].at[pl.ds(0, n)], out_ref.at[pl.ds(dr, n)], wr_sems.at[b])
        if c + _NBUF < n_chunks:
            wr[c].wait()
            start_read(c + _NBUF)

    pads = []
    for i, r0, rows in pad_segs:
        r = r0
        while r < r0 + rows:
            n = min(_PCH, r0 + rows - r)
            pads.append(pltpu.async_copy(
                zsh.at[pl.ds(0, n)],
                out_ref.at[pl.ds(i * _MAX_L + r, n)], pad_sem))
            r += n
    for h in pads:
        h.wait()

    for c in range(max(0, n_chunks - _NBUF), n_chunks):
        if wr[c] is not None and c + _NBUF >= n_chunks:
            wr[c].wait()


def _sc_body(s0, s1, s2, s3, s4, s5, s6, s7, zsrc, out_ref,
             b0, b1, b2, zsh, rd_sems, wr_sems, pad_sem, z_sem):
    in_refs = (s0, s1, s2, s3, s4, s5, s6, s7)
    bufs = (b0, b1, b2)
    cid = lax.axis_index("c")
    sid = lax.axis_index("s")
    wid = sid * _NC + cid

    @pl.when(sid == 0)
    def _():
        pltpu.async_copy(zsrc, zsh, z_sem).wait()

    plsc.subcore_barrier()

    for w in range(_NW):
        @pl.when(wid == w)
        def _(w=w):
            _emit_worker(w, in_refs, out_ref, bufs, zsh,
                         rd_sems, wr_sems, pad_sem)


def kernel(seq0, seq1, seq2, seq3, seq4, seq5, seq6, seq7):
    seqs = [s.reshape(s.shape[1], _D) for s in
            (seq0, seq1, seq2, seq3, seq4, seq5, seq6, seq7)]
    zsrc = jnp.zeros((_PCH, _D), jnp.float32)
    mesh = plsc.VectorSubcoreMesh(core_axis_name="c", subcore_axis_name="s")
    run = pl.kernel(
        _sc_body,
        out_type=jax.ShapeDtypeStruct((_TOTAL_ROWS, _D), jnp.float32),
        mesh=mesh,
        scratch_types=[
            pltpu.VMEM((_CH, _D), jnp.float32),
            pltpu.VMEM((_CH, _D), jnp.float32),
            pltpu.VMEM((_CH, _D), jnp.float32),
            pltpu.VMEM_SHARED((_PCH, _D), jnp.float32),
            pltpu.SemaphoreType.DMA((_NBUF,)),
            pltpu.SemaphoreType.DMA((_NBUF,)),
            pltpu.SemaphoreType.DMA,
            pltpu.SemaphoreType.DMA,
        ],
    )
    out = run(*seqs, zsrc)
    return out.reshape(8, _MAX_L, _D)
